# Initial kernel scaffold; baseline (speedup 1.0000x reference)
import jax
import jax.numpy as jnp
from jax.experimental import pallas as pl


def kernel(x, edge_index, W1, b1, W2, b2):
    raise NotImplementedError("write your pallas kernel here")


# speedup vs baseline: 6.0301x; 6.0301x over previous
import functools

import jax
import jax.numpy as jnp
from jax import lax
from jax.experimental import pallas as pl
from jax.experimental.pallas import tpu as pltpu
from jax.experimental.pallas import tpu_sc as plsc

N = 10000
E = 160000
IN_DIM = 256
HID_DIM = 512

NP = 10240
EP = 163840
EB = EP // 128
NC = 2
NS = 16
ROWS_PER_TILE = NP // NS
EROWS_SC = EB // NS
EROWS_ALL = EB // (NC * NS)

_MESH = plsc.VectorSubcoreMesh(
    core_axis_name="c", subcore_axis_name="s", num_cores=NC, num_subcores=NS)


def _deg_body(dst_hbm, zeros_hbm, ones_hbm, out_hbm, dstb, onesb, accd):
    c = lax.axis_index("c")
    s = lax.axis_index("s")
    w = s * NC + c
    base = s * ROWS_PER_TILE
    pltpu.sync_copy(zeros_hbm.at[pl.ds(base, ROWS_PER_TILE)],
                    accd.at[pl.ds(base, ROWS_PER_TILE)])
    pltpu.sync_copy(ones_hbm, onesb)
    pltpu.sync_copy(dst_hbm.at[pl.ds(w * EROWS_ALL, EROWS_ALL)], dstb)
    plsc.subcore_barrier()

    def step(j, carry):
        pltpu.sync_copy(onesb, accd.at[dstb.at[j]], add=True)
        return carry

    lax.fori_loop(0, EROWS_ALL, step, 0)
    plsc.subcore_barrier()
    pltpu.sync_copy(accd.at[pl.ds(base, ROWS_PER_TILE)],
                    out_hbm.at[c, pl.ds(base, ROWS_PER_TILE)])


_deg_kernel = pl.kernel(
    _deg_body,
    out_type=jax.ShapeDtypeStruct((NC, NP, 16), jnp.float32),
    mesh=_MESH,
    scratch_types=[
        pltpu.VMEM((EROWS_ALL, 128), jnp.int32),
        pltpu.VMEM((128, 16), jnp.float32),
        pltpu.VMEM_SHARED((NP, 16), jnp.float32),
    ],
)


def _make_agg_kernel(n_chunks):
    per_sc = n_chunks // NC

    def _do_chunk(chunk, hs_hbm, out_hbm, srcb, dstb, rows, sem, acc, s):
        base = s * ROWS_PER_TILE
        pltpu.sync_copy(hs_hbm.at[chunk, pl.ds(base, ROWS_PER_TILE)],
                        acc.at[pl.ds(base, ROWS_PER_TILE)])
        plsc.subcore_barrier()

        def step(j, carry):
            pltpu.async_copy(hs_hbm.at[chunk].at[srcb.at[j]], rows, sem).wait()
            pltpu.sync_copy(rows, acc.at[dstb.at[j]], add=True)
            return carry

        lax.fori_loop(0, EROWS_SC, step, 0)
        plsc.subcore_barrier()
        pltpu.sync_copy(acc.at[pl.ds(base, ROWS_PER_TILE)],
                        out_hbm.at[chunk, pl.ds(base, ROWS_PER_TILE)])
        plsc.subcore_barrier()

    def body(hs_hbm, src_hbm, dst_hbm, out_hbm, srcb, dstb, rows, sem, acc):
        core = lax.axis_index("c")
        s = lax.axis_index("s")
        pltpu.sync_copy(src_hbm.at[pl.ds(s * EROWS_SC, EROWS_SC)], srcb)
        pltpu.sync_copy(dst_hbm.at[pl.ds(s * EROWS_SC, EROWS_SC)], dstb)
        for ci in range(per_sc):
            for co in range(NC):
                @pl.when(core == co)
                def _():
                    _do_chunk(co * per_sc + ci, hs_hbm, out_hbm,
                              srcb, dstb, rows, sem, acc, s)

    return pl.kernel(
        body,
        out_type=jax.ShapeDtypeStruct((n_chunks, NP, 128), jnp.float32),
        mesh=_MESH,
        scratch_types=[
            pltpu.VMEM((EROWS_SC, 128), jnp.int32),
            pltpu.VMEM((EROWS_SC, 128), jnp.int32),
            pltpu.VMEM((128, 128), jnp.float32),
            pltpu.SemaphoreType.DMA,
            pltpu.VMEM_SHARED((NP, 128), jnp.float32),
        ],
    )


_agg4 = _make_agg_kernel(4)
_agg2 = _make_agg_kernel(2)


TN = 512


def _dis(degp_ref):
    deg = degp_ref[0, :, 0] + degp_ref[1, :, 0] + 1.0
    return lax.rsqrt(deg)


def _mm1_body(x_ref, w_ref, degp_ref, out_ref):
    dis = _dis(degp_ref)
    h = jnp.dot(x_ref[...], w_ref[...], preferred_element_type=jnp.float32)
    out_ref[0] = h * dis[:, None]


def _mm1(xp, W1, degp):
    return pl.pallas_call(
        _mm1_body,
        grid=(NP // TN, HID_DIM // 128),
        in_specs=[
            pl.BlockSpec((TN, IN_DIM), lambda i, c: (i, 0)),
            pl.BlockSpec((IN_DIM, 128), lambda i, c: (0, c)),
            pl.BlockSpec((NC, TN, 16), lambda i, c: (0, i, 0)),
        ],
        out_specs=pl.BlockSpec((1, TN, 128), lambda i, c: (c, i, 0)),
        out_shape=jax.ShapeDtypeStruct((HID_DIM // 128, NP, 128), jnp.float32),
    )(xp, W1, degp)


def _mm2_body(agg_ref, w2_ref, b1_ref, degp_ref, out_ref):
    dis = _dis(degp_ref)
    acc = jnp.zeros((TN, IN_DIM), jnp.float32)
    for c in range(HID_DIM // 128):
        z = jax.nn.relu(agg_ref[c] * dis[:, None] + b1_ref[c][None, :])
        acc = acc + jnp.dot(z, w2_ref[c], preferred_element_type=jnp.float32)
    h2s = acc * dis[:, None]
    out_ref[0] = h2s[:, :128]
    out_ref[1] = h2s[:, 128:]


def _mm2(agg1, w2r, b1r, degp):
    return pl.pallas_call(
        _mm2_body,
        grid=(NP // TN,),
        in_specs=[
            pl.BlockSpec((HID_DIM // 128, TN, 128), lambda i: (0, i, 0)),
            pl.BlockSpec((HID_DIM // 128, 128, IN_DIM), lambda i: (0, 0, 0)),
            pl.BlockSpec((HID_DIM // 128, 128), lambda i: (0, 0)),
            pl.BlockSpec((NC, TN, 16), lambda i: (0, i, 0)),
        ],
        out_specs=pl.BlockSpec((2, TN, 128), lambda i: (0, i, 0)),
        out_shape=jax.ShapeDtypeStruct((IN_DIM // 128, NP, 128), jnp.float32),
    )(agg1, w2r, b1r, degp)


def _final_body(agg_ref, b2_ref, degp_ref, out_ref):
    dis = _dis(degp_ref)
    out_ref[:, :128] = agg_ref[0] * dis[:, None] + b2_ref[0][None, :]
    out_ref[:, 128:] = agg_ref[1] * dis[:, None] + b2_ref[1][None, :]


def _final(agg2, b2r, degp):
    return pl.pallas_call(
        _final_body,
        grid=(NP // TN,),
        in_specs=[
            pl.BlockSpec((IN_DIM // 128, TN, 128), lambda i: (0, i, 0)),
            pl.BlockSpec((IN_DIM // 128, 128), lambda i: (0, 0)),
            pl.BlockSpec((NC, TN, 16), lambda i: (0, i, 0)),
        ],
        out_specs=pl.BlockSpec((TN, IN_DIM), lambda i: (i, 0)),
        out_shape=jax.ShapeDtypeStruct((NP, IN_DIM), jnp.float32),
    )(agg2, b2r, degp)


def kernel(x, edge_index, W1, b1, W2, b2):
    ei = edge_index.astype(jnp.int32)
    pad = EP - E
    srcp = jnp.concatenate([ei[0], jnp.zeros((pad,), jnp.int32)])
    dstp = jnp.concatenate([ei[1], jnp.full((pad,), N, jnp.int32)])
    src2d = srcp.reshape(EB, 128)
    dst2d = dstp.reshape(EB, 128)
    xp = jnp.pad(x, ((0, NP - N), (0, 0)))
    w2r = W2.reshape(HID_DIM // 128, 128, IN_DIM)
    b1r = b1.reshape(HID_DIM // 128, 128)
    b2r = b2.reshape(IN_DIM // 128, 128)
    zeros16 = jnp.zeros((NP, 16), jnp.float32)
    ones16 = jnp.ones((128, 16), jnp.float32)

    degp = _deg_kernel(dst2d, zeros16, ones16)
    hs1 = _mm1(xp, W1, degp)
    agg1 = _agg4(hs1, src2d, dst2d)
    hs2 = _mm2(agg1, w2r, b1r, degp)
    agg2 = _agg2(hs2, src2d, dst2d)
    outp = _final(agg2, b2r, degp)
    return outp[:N]


# speedup vs baseline: 6.0304x; 1.0000x over previous
import jax
import jax.numpy as jnp
from jax import lax
from jax.experimental import pallas as pl
from jax.experimental.pallas import tpu as pltpu
from jax.experimental.pallas import tpu_sc as plsc

N = 10000
E = 160000
IN_DIM = 256
HID_DIM = 512

NP = 10240
EP = 163840
EB = EP // 128
NC = 2
NS = 16
ROWS_PER_TILE = NP // NS
EROWS_SC = EB // NS
EROWS_ALL = EB // (NC * NS)

_MESH = plsc.VectorSubcoreMesh(
    core_axis_name="c", subcore_axis_name="s", num_cores=NC, num_subcores=NS)

_SEM_PAD = [pltpu.SemaphoreType.DMA] * 16


def _deg_body(dst_hbm, zeros_hbm, ones_hbm, out_hbm, *args):
    dstb, onesb, accd = args[16:]
    c = lax.axis_index("c")
    s = lax.axis_index("s")
    w = s * NC + c
    base = s * ROWS_PER_TILE
    pltpu.sync_copy(zeros_hbm.at[pl.ds(base, ROWS_PER_TILE)],
                    accd.at[pl.ds(base, ROWS_PER_TILE)])
    pltpu.sync_copy(ones_hbm, onesb)
    pltpu.sync_copy(dst_hbm.at[pl.ds(w * EROWS_ALL, EROWS_ALL)], dstb)
    plsc.subcore_barrier()

    def step(j, carry):
        pltpu.sync_copy(onesb, accd.at[dstb.at[j]], add=True)
        return carry

    lax.fori_loop(0, EROWS_ALL, step, 0)
    plsc.subcore_barrier()
    pltpu.sync_copy(accd.at[pl.ds(base, ROWS_PER_TILE)],
                    out_hbm.at[c, pl.ds(base, ROWS_PER_TILE)])


_deg_kernel = pl.kernel(
    _deg_body,
    out_type=jax.ShapeDtypeStruct((NC, NP, 16), jnp.float32),
    mesh=_MESH,
    scratch_types=_SEM_PAD + [
        pltpu.VMEM((EROWS_ALL, 128), jnp.int32),
        pltpu.VMEM((128, 16), jnp.float32),
        pltpu.VMEM_SHARED((NP, 16), jnp.float32),
    ],
)


def _make_agg_kernel(n_chunks):
    per_sc = n_chunks // NC

    def _do_chunk(chunk, hs_hbm, out_hbm, srcb, dstb, rows, sem, acc, s):
        base = s * ROWS_PER_TILE
        pltpu.sync_copy(hs_hbm.at[chunk, pl.ds(base, ROWS_PER_TILE)],
                        acc.at[pl.ds(base, ROWS_PER_TILE)])
        plsc.subcore_barrier()

        def step(j, carry):
            pltpu.async_copy(hs_hbm.at[chunk].at[srcb.at[j]], rows, sem).wait()
            pltpu.sync_copy(rows, acc.at[dstb.at[j]], add=True)
            return carry

        lax.fori_loop(0, EROWS_SC, step, 0)
        plsc.subcore_barrier()
        pltpu.sync_copy(acc.at[pl.ds(base, ROWS_PER_TILE)],
                        out_hbm.at[chunk, pl.ds(base, ROWS_PER_TILE)])
        plsc.subcore_barrier()

    def body(hs_hbm, src_hbm, dst_hbm, out_hbm, *args):
        srcb, dstb, rows, sem, acc = args[16:]
        core = lax.axis_index("c")
        s = lax.axis_index("s")
        pltpu.sync_copy(src_hbm.at[pl.ds(s * EROWS_SC, EROWS_SC)], srcb)
        pltpu.sync_copy(dst_hbm.at[pl.ds(s * EROWS_SC, EROWS_SC)], dstb)
        for ci in range(per_sc):
            for co in range(NC):
                @pl.when(core == co)
                def _():
                    _do_chunk(co * per_sc + ci, hs_hbm, out_hbm,
                              srcb, dstb, rows, sem, acc, s)

    return pl.kernel(
        body,
        out_type=jax.ShapeDtypeStruct((n_chunks, NP, 128), jnp.float32),
        mesh=_MESH,
        scratch_types=_SEM_PAD + [
            pltpu.VMEM((EROWS_SC, 128), jnp.int32),
            pltpu.VMEM((EROWS_SC, 128), jnp.int32),
            pltpu.VMEM((128, 128), jnp.float32),
            pltpu.SemaphoreType.DMA,
            pltpu.VMEM_SHARED((NP, 128), jnp.float32),
        ],
    )


_agg4 = _make_agg_kernel(4)
_agg2 = _make_agg_kernel(2)


TN = 512


def _dis(degp_ref):
    deg = degp_ref[0, :, 0] + degp_ref[1, :, 0] + 1.0
    return lax.rsqrt(deg)


def _mm1_body(x_ref, w_ref, degp_ref, out_ref):
    dis = _dis(degp_ref)
    h = jnp.dot(x_ref[...], w_ref[...], preferred_element_type=jnp.float32)
    out_ref[0] = h * dis[:, None]


def _mm1(xp, W1, degp):
    return pl.pallas_call(
        _mm1_body,
        grid=(NP // TN, HID_DIM // 128),
        in_specs=[
            pl.BlockSpec((TN, IN_DIM), lambda i, c: (i, 0)),
            pl.BlockSpec((IN_DIM, 128), lambda i, c: (0, c)),
            pl.BlockSpec((NC, TN, 16), lambda i, c: (0, i, 0)),
        ],
        out_specs=pl.BlockSpec((1, TN, 128), lambda i, c: (c, i, 0)),
        out_shape=jax.ShapeDtypeStruct((HID_DIM // 128, NP, 128), jnp.float32),
    )(xp, W1, degp)


def _mm2_body(agg_ref, w2_ref, b1_ref, degp_ref, out_ref):
    dis = _dis(degp_ref)
    acc = jnp.zeros((TN, IN_DIM), jnp.float32)
    for c in range(HID_DIM // 128):
        z = jax.nn.relu(agg_ref[c] * dis[:, None] + b1_ref[c][None, :])
        acc = acc + jnp.dot(z, w2_ref[c], preferred_element_type=jnp.float32)
    h2s = acc * dis[:, None]
    out_ref[0] = h2s[:, :128]
    out_ref[1] = h2s[:, 128:]


def _mm2(agg1, w2r, b1r, degp):
    return pl.pallas_call(
        _mm2_body,
        grid=(NP // TN,),
        in_specs=[
            pl.BlockSpec((HID_DIM // 128, TN, 128), lambda i: (0, i, 0)),
            pl.BlockSpec((HID_DIM // 128, 128, IN_DIM), lambda i: (0, 0, 0)),
            pl.BlockSpec((HID_DIM // 128, 128), lambda i: (0, 0)),
            pl.BlockSpec((NC, TN, 16), lambda i: (0, i, 0)),
        ],
        out_specs=pl.BlockSpec((2, TN, 128), lambda i: (0, i, 0)),
        out_shape=jax.ShapeDtypeStruct((IN_DIM // 128, NP, 128), jnp.float32),
    )(agg1, w2r, b1r, degp)


def _final_body(agg_ref, b2_ref, degp_ref, out_ref):
    dis = _dis(degp_ref)
    out_ref[:, :128] = agg_ref[0] * dis[:, None] + b2_ref[0][None, :]
    out_ref[:, 128:] = agg_ref[1] * dis[:, None] + b2_ref[1][None, :]


def _final(agg2, b2r, degp):
    return pl.pallas_call(
        _final_body,
        grid=(NP // TN,),
        in_specs=[
            pl.BlockSpec((IN_DIM // 128, TN, 128), lambda i: (0, i, 0)),
            pl.BlockSpec((IN_DIM // 128, 128), lambda i: (0, 0)),
            pl.BlockSpec((NC, TN, 16), lambda i: (0, i, 0)),
        ],
        out_specs=pl.BlockSpec((TN, IN_DIM), lambda i: (i, 0)),
        out_shape=jax.ShapeDtypeStruct((NP, IN_DIM), jnp.float32),
    )(agg2, b2r, degp)


def kernel(x, edge_index, W1, b1, W2, b2):
    ei = edge_index.astype(jnp.int32)
    pad = EP - E
    srcp = jnp.concatenate([ei[0], jnp.zeros((pad,), jnp.int32)])
    dstp = jnp.concatenate([ei[1], jnp.full((pad,), N, jnp.int32)])
    src2d = srcp.reshape(EB, 128)
    dst2d = dstp.reshape(EB, 128)
    xp = jnp.pad(x, ((0, NP - N), (0, 0)))
    w2r = W2.reshape(HID_DIM // 128, 128, IN_DIM)
    b1r = b1.reshape(HID_DIM // 128, 128)
    b2r = b2.reshape(IN_DIM // 128, 128)
    zeros16 = jnp.zeros((NP, 16), jnp.float32)
    ones16 = jnp.ones((128, 16), jnp.float32)

    degp = _deg_kernel(dst2d, zeros16, ones16)
    hs1 = _mm1(xp, W1, degp)
    agg1 = _agg4(hs1, src2d, dst2d)
    hs2 = _mm2(agg1, w2r, b1r, degp)
    agg2 = _agg2(hs2, src2d, dst2d)
    outp = _final(agg2, b2r, degp)
    return outp[:N]
